# baseline (speedup 1.0000x reference)
import functools

import jax
import jax.numpy as jnp
from jax import lax
from jax.experimental import pallas as pl
from jax.experimental.pallas import tpu as pltpu
from jax.experimental.pallas import tpu_sc as plsc

N_NODES = 10000
IN_DIM = 128
HID_DIM = 128
OUT_DIM = 16

NC = 2
NS = 16
NW = NC * NS
STRIPE = 640
NPAD = NS * STRIPE
BR = 2000


def _fill(ref, rows, cols, value):
    groups = cols // 16

    def body(t, carry):
        ref[t // groups, pl.ds((t % groups) * 16, 16)] = jnp.full(
            (16,), value, jnp.float32)
        return carry

    lax.fori_loop(0, rows * groups, body, 0)


def _zero_stripe(zref, rows, acc, s):
    d = zref.shape[-1]
    groups = d // 16

    def body(t, carry):
        zref[t // groups, pl.ds((t % groups) * 16, 16)] = jnp.zeros(
            (16,), jnp.float32)
        return carry

    lax.fori_loop(0, rows * groups, body, 0)
    for kk in range(STRIPE // rows):
        pltpu.sync_copy(zref, acc.at[pl.ds(s * STRIPE + kk * rows, rows)])


def _drain_stripe(acc, out_hbm, c, s, rows):
    for kk in range(STRIPE // rows):
        off = s * STRIPE + kk * rows
        pltpu.sync_copy(acc.at[pl.ds(off, rows)],
                        out_hbm.at[c, pl.ds(off, rows)])


def _edge_pipeline(z_hbm, src_v, dst_v, bufs, gsem, acc, n_blocks, nbuf):
    for b in range(nbuf):
        pltpu.async_copy(z_hbm.at[src_v.at[b]], bufs.at[b], gsem[b])

    def body(g, carry):
        for b in range(nbuf):
            j = g * nbuf + b
            pltpu.make_async_copy(
                z_hbm.at[src_v.at[j]], bufs.at[b], gsem[b]).wait()
            pltpu.sync_copy(bufs.at[b], acc.at[dst_v.at[j]], add=True)

            @pl.when(j + nbuf < n_blocks)
            def _():
                pltpu.async_copy(
                    z_hbm.at[src_v.at[j + nbuf]], bufs.at[b], gsem[b])
        return carry

    lax.fori_loop(0, n_blocks // nbuf, body, 0)


def _make_deg_kernel(n_blocks, k):
    mesh = plsc.VectorSubcoreMesh(core_axis_name="c", subcore_axis_name="s")

    @functools.partial(
        pl.kernel,
        mesh=mesh,
        out_type=jax.ShapeDtypeStruct((NC, NPAD, 16), jnp.float32),
        scratch_types=[
            pltpu.VMEM((n_blocks, k), jnp.int32),
            pltpu.VMEM((k, 16), jnp.float32),
            pltpu.VMEM_SHARED((NPAD, 16), jnp.float32),
        ],
        compiler_params=pltpu.CompilerParams(use_tc_tiling_on_sc=False),
    )
    def deg_kernel(dst_hbm, out_hbm, dst_v, ones_v, acc):
        c = lax.axis_index("c")
        s = lax.axis_index("s")
        wid = c * NS + s
        _zero_stripe(ones_v, k, acc, s)
        _fill(ones_v, k, 16, 1.0)
        pltpu.sync_copy(dst_hbm.at[wid], dst_v)
        plsc.subcore_barrier()

        def body(j, carry):
            pltpu.sync_copy(ones_v, acc.at[dst_v.at[j]], add=True)
            return carry

        lax.fori_loop(0, n_blocks, body, 0)
        plsc.subcore_barrier()
        _drain_stripe(acc, out_hbm, c, s, k)

    return deg_kernel


SLOW_CORE = 1


def _make_agg_kernel(nb_fast, nb_slow, d, k, nbuf, phases=1):
    nb_f = nb_fast // phases
    nb_s = nb_slow // phases
    assert nb_fast % phases == 0 and nb_slow % phases == 0
    assert nb_f % nbuf == 0 and nb_s % nbuf == 0 and STRIPE % k == 0
    nb_max = max(nb_f, nb_s)
    mesh = plsc.VectorSubcoreMesh(core_axis_name="c", subcore_axis_name="s")

    @functools.partial(
        pl.kernel,
        mesh=mesh,
        out_type=jax.ShapeDtypeStruct((NC, NPAD, d), jnp.float32),
        scratch_types=[
            pltpu.VMEM((nb_max, k), jnp.int32),
            pltpu.VMEM((nb_max, k), jnp.int32),
            pltpu.VMEM((nbuf, k, d), jnp.float32),
            pltpu.VMEM_SHARED((NPAD, d), jnp.float32),
        ] + [pltpu.SemaphoreType.DMA] * nbuf,
        compiler_params=pltpu.CompilerParams(use_tc_tiling_on_sc=False),
    )
    def agg_kernel(z_hbm, srcf_hbm, dstf_hbm, srcs_hbm, dsts_hbm, out_hbm,
                   src_v, dst_v, bufs, acc, *gsem):
        c = lax.axis_index("c")
        s = lax.axis_index("s")
        n_my = jnp.where(c == SLOW_CORE, nb_s, nb_f)
        _zero_stripe(bufs.at[0], k, acc, s)
        plsc.subcore_barrier()

        for p in range(phases):
            @pl.when(c != SLOW_CORE)
            def _():
                pltpu.sync_copy(srcf_hbm.at[s, pl.ds(p * nb_f, nb_f)],
                                src_v.at[pl.ds(0, nb_f)])
                pltpu.sync_copy(dstf_hbm.at[s, pl.ds(p * nb_f, nb_f)],
                                dst_v.at[pl.ds(0, nb_f)])

            @pl.when(c == SLOW_CORE)
            def _():
                pltpu.sync_copy(srcs_hbm.at[s, pl.ds(p * nb_s, nb_s)],
                                src_v.at[pl.ds(0, nb_s)])
                pltpu.sync_copy(dsts_hbm.at[s, pl.ds(p * nb_s, nb_s)],
                                dst_v.at[pl.ds(0, nb_s)])

            _edge_pipeline(z_hbm, src_v, dst_v, bufs, gsem, acc, n_my, nbuf)

        plsc.subcore_barrier()
        _drain_stripe(acc, out_hbm, c, s, k)

    return agg_kernel


def _dinv_of(d0, d1):
    return lax.rsqrt(1.0 + d0[0][:, :1] + d1[0][:, :1])


def _mm1_body(x_ref, d0_ref, d1_ref, w_ref, o_ref):
    dinv = _dinv_of(d0_ref[...], d1_ref[...])
    y = jnp.dot(x_ref[...], w_ref[...], preferred_element_type=jnp.float32)
    o_ref[...] = y * dinv


def _fuse_body(p0_ref, p1_ref, z1_ref, d0_ref, d1_ref, b1_ref, w2_ref, o_ref):
    dinv = _dinv_of(d0_ref[...], d1_ref[...])
    h = dinv * (p0_ref[0] + p1_ref[0] + z1_ref[...]) + b1_ref[...]
    h = jnp.maximum(h, 0.0)
    y2 = jnp.dot(h, w2_ref[...], preferred_element_type=jnp.float32)
    o_ref[...] = y2 * dinv


def _fin_body(q0_ref, q1_ref, z2_ref, d0_ref, d1_ref, b2_ref, o_ref):
    dinv = _dinv_of(d0_ref[...], d1_ref[...])
    o_ref[...] = dinv * (q0_ref[0] + q1_ref[0] + z2_ref[...]) + b2_ref[...]


def _row_spec(cols):
    return pl.BlockSpec((BR, cols), lambda i: (i, 0))


def _part_spec(cols, which):
    if which == 0:
        return pl.BlockSpec((1, BR, cols), lambda i: (0, i, 0))
    return pl.BlockSpec((1, BR, cols), lambda i: (1, i, 0))


def _full_spec(rows, cols):
    return pl.BlockSpec((rows, cols), lambda i: (0, 0))


def _split_views(flat, nb_fast, nb_slow, k):
    cut = NS * nb_fast * k
    fast = flat[:cut].reshape(NS, nb_fast, k)
    slow = flat[cut:cut + NS * nb_slow * k].reshape(NS, nb_slow, k)
    return fast, slow


def kernel(x, edge_index, W1, b1, W2, b2):
    src = edge_index[0].astype(jnp.int32)
    dst = edge_index[1].astype(jnp.int32)
    e = src.shape[0]

    K1, NBUF1 = 64, 3
    NB1_FAST, NB1_SLOW = 234, 84
    K2, NBUF2 = 128, 4
    NB2_FAST, NB2_SLOW = 132, 28
    nbd = -(-e // (NW * K2))
    slots = NS * (NB2_FAST + NB2_SLOW) * K2
    src_p = jnp.concatenate([src, jnp.zeros((slots - e,), jnp.int32)])
    dst_p = jnp.concatenate([dst, jnp.full((slots - e,), N_NODES, jnp.int32)])
    src_1f, src_1s = _split_views(src_p, NB1_FAST, NB1_SLOW, K1)
    dst_1f, dst_1s = _split_views(dst_p, NB1_FAST, NB1_SLOW, K1)
    src_2f, src_2s = _split_views(src_p, NB2_FAST, NB2_SLOW, K2)
    dst_2f, dst_2s = _split_views(dst_p, NB2_FAST, NB2_SLOW, K2)
    dst_d = dst_p[:NW * nbd * K2].reshape(NW, nbd, K2)

    degp = _make_deg_kernel(nbd, K2)(dst_d)

    grid = (N_NODES // BR,)
    z1 = pl.pallas_call(
        _mm1_body,
        grid=grid,
        in_specs=[_row_spec(IN_DIM), _part_spec(16, 0), _part_spec(16, 1),
                  _full_spec(IN_DIM, HID_DIM)],
        out_specs=_row_spec(HID_DIM),
        out_shape=jax.ShapeDtypeStruct((N_NODES, HID_DIM), jnp.float32),
    )(x, degp, degp, W1)

    agg1 = _make_agg_kernel(NB1_FAST, NB1_SLOW, HID_DIM, K1, NBUF1, phases=2)(
        z1, src_1f, dst_1f, src_1s, dst_1s)

    z2 = pl.pallas_call(
        _fuse_body,
        grid=grid,
        in_specs=[_part_spec(HID_DIM, 0), _part_spec(HID_DIM, 1),
                  _row_spec(HID_DIM), _part_spec(16, 0), _part_spec(16, 1),
                  _full_spec(1, HID_DIM), _full_spec(HID_DIM, OUT_DIM)],
        out_specs=_row_spec(OUT_DIM),
        out_shape=jax.ShapeDtypeStruct((N_NODES, OUT_DIM), jnp.float32),
    )(agg1, agg1, z1, degp, degp, b1.reshape(1, HID_DIM), W2)

    agg2 = _make_agg_kernel(NB2_FAST, NB2_SLOW, OUT_DIM, K2, NBUF2)(
        z2, src_2f, dst_2f, src_2s, dst_2s)

    out = pl.pallas_call(
        _fin_body,
        grid=grid,
        in_specs=[_part_spec(OUT_DIM, 0), _part_spec(OUT_DIM, 1),
                  _row_spec(OUT_DIM), _part_spec(16, 0), _part_spec(16, 1),
                  _full_spec(1, OUT_DIM)],
        out_specs=_row_spec(OUT_DIM),
        out_shape=jax.ShapeDtypeStruct((N_NODES, OUT_DIM), jnp.float32),
    )(agg2, agg2, z2, degp, degp, b2.reshape(1, OUT_DIM))

    return out


# speedup vs baseline: 1.2519x; 1.2519x over previous
import functools

import jax
import jax.numpy as jnp
from jax import lax
from jax.experimental import pallas as pl
from jax.experimental.pallas import tpu as pltpu
from jax.experimental.pallas import tpu_sc as plsc

N_NODES = 10000
IN_DIM = 128
HID_DIM = 128
OUT_DIM = 16

NC = 2
NS = 16
NW = NC * NS
STRIPE = 640
NPAD = NS * STRIPE
BR = 2000


def _fill(ref, rows, cols, value):
    groups = cols // 16

    def body(t, carry):
        ref[t // groups, pl.ds((t % groups) * 16, 16)] = jnp.full(
            (16,), value, jnp.float32)
        return carry

    lax.fori_loop(0, rows * groups, body, 0)


def _zero_stripe(zref, rows, acc, s):
    d = zref.shape[-1]
    groups = d // 16

    def body(t, carry):
        zref[t // groups, pl.ds((t % groups) * 16, 16)] = jnp.zeros(
            (16,), jnp.float32)
        return carry

    lax.fori_loop(0, rows * groups, body, 0)
    for kk in range(STRIPE // rows):
        pltpu.sync_copy(zref, acc.at[pl.ds(s * STRIPE + kk * rows, rows)])


def _drain_stripe(acc, out_hbm, c, s, rows):
    for kk in range(STRIPE // rows):
        off = s * STRIPE + kk * rows
        pltpu.sync_copy(acc.at[pl.ds(off, rows)],
                        out_hbm.at[c, pl.ds(off, rows)])


def _edge_pipeline(z_hbm, src_v, dst_v, bufs, gsem, acc, n_blocks, nbuf):
    for b in range(nbuf):
        pltpu.async_copy(z_hbm.at[src_v.at[b]], bufs.at[b], gsem[b])

    def body(g, carry):
        for b in range(nbuf):
            j = g * nbuf + b
            pltpu.make_async_copy(
                z_hbm.at[src_v.at[j]], bufs.at[b], gsem[b]).wait()
            pltpu.sync_copy(bufs.at[b], acc.at[dst_v.at[j]], add=True)

            @pl.when(j + nbuf < n_blocks)
            def _():
                pltpu.async_copy(
                    z_hbm.at[src_v.at[j + nbuf]], bufs.at[b], gsem[b])
        return carry

    lax.fori_loop(0, n_blocks // nbuf, body, 0)


def _make_deg_kernel(n_blocks, k):
    mesh = plsc.VectorSubcoreMesh(core_axis_name="c", subcore_axis_name="s")

    @functools.partial(
        pl.kernel,
        mesh=mesh,
        out_type=jax.ShapeDtypeStruct((NC, NPAD, 16), jnp.float32),
        scratch_types=[
            pltpu.VMEM((n_blocks, k), jnp.int32),
            pltpu.VMEM((k, 16), jnp.float32),
            pltpu.VMEM_SHARED((NPAD, 16), jnp.float32),
        ],
        compiler_params=pltpu.CompilerParams(use_tc_tiling_on_sc=False),
    )
    def deg_kernel(dst_hbm, out_hbm, dst_v, ones_v, acc):
        c = lax.axis_index("c")
        s = lax.axis_index("s")
        wid = c * NS + s
        _zero_stripe(ones_v, k, acc, s)
        _fill(ones_v, k, 16, 1.0)
        pltpu.sync_copy(dst_hbm.at[wid], dst_v)
        plsc.subcore_barrier()

        def body(j, carry):
            pltpu.sync_copy(ones_v, acc.at[dst_v.at[j]], add=True)
            return carry

        lax.fori_loop(0, n_blocks, body, 0)
        plsc.subcore_barrier()
        _drain_stripe(acc, out_hbm, c, s, k)

    return deg_kernel


SLOW_CORE = 1


def _make_agg_kernel(nb_fast, nb_slow, d, k, nbuf, nbuf_slow=None, phases=1):
    nbuf_slow = nbuf if nbuf_slow is None else nbuf_slow
    nb_f = nb_fast // phases
    nb_s = nb_slow // phases
    assert nb_fast % phases == 0 and nb_slow % phases == 0
    assert nb_f % nbuf == 0 and nb_s % nbuf_slow == 0 and STRIPE % k == 0
    nb_max = max(nb_f, nb_s)
    mesh = plsc.VectorSubcoreMesh(core_axis_name="c", subcore_axis_name="s")

    @functools.partial(
        pl.kernel,
        mesh=mesh,
        out_type=jax.ShapeDtypeStruct((NC, NPAD, d), jnp.float32),
        scratch_types=[
            pltpu.VMEM((nb_max, k), jnp.int32),
            pltpu.VMEM((nb_max, k), jnp.int32),
            pltpu.VMEM((nbuf, k, d), jnp.float32),
            pltpu.VMEM_SHARED((NPAD, d), jnp.float32),
        ] + [pltpu.SemaphoreType.DMA] * nbuf,
        compiler_params=pltpu.CompilerParams(use_tc_tiling_on_sc=False),
    )
    def agg_kernel(z_hbm, srcf_hbm, dstf_hbm, srcs_hbm, dsts_hbm, out_hbm,
                   src_v, dst_v, bufs, acc, *gsem):
        c = lax.axis_index("c")
        s = lax.axis_index("s")
        _zero_stripe(bufs.at[0], k, acc, s)
        plsc.subcore_barrier()

        for p in range(phases):
            @pl.when(c != SLOW_CORE)
            def _():
                pltpu.sync_copy(srcf_hbm.at[s, pl.ds(p * nb_f, nb_f)],
                                src_v.at[pl.ds(0, nb_f)])
                pltpu.sync_copy(dstf_hbm.at[s, pl.ds(p * nb_f, nb_f)],
                                dst_v.at[pl.ds(0, nb_f)])
                _edge_pipeline(z_hbm, src_v, dst_v, bufs, gsem, acc,
                               nb_f, nbuf)

            @pl.when(c == SLOW_CORE)
            def _():
                pltpu.sync_copy(srcs_hbm.at[s, pl.ds(p * nb_s, nb_s)],
                                src_v.at[pl.ds(0, nb_s)])
                pltpu.sync_copy(dsts_hbm.at[s, pl.ds(p * nb_s, nb_s)],
                                dst_v.at[pl.ds(0, nb_s)])
                _edge_pipeline(z_hbm, src_v, dst_v, bufs, gsem, acc,
                               nb_s, nbuf_slow)

        plsc.subcore_barrier()
        _drain_stripe(acc, out_hbm, c, s, k)

    return agg_kernel


def _dinv_of(d0, d1):
    return lax.rsqrt(1.0 + d0[0][:, :1] + d1[0][:, :1])


def _mm1_body(x_ref, d0_ref, d1_ref, w_ref, o_ref):
    dinv = _dinv_of(d0_ref[...], d1_ref[...])
    y = jnp.dot(x_ref[...], w_ref[...], preferred_element_type=jnp.float32)
    o_ref[...] = y * dinv


def _fuse_body(p0_ref, p1_ref, z1_ref, d0_ref, d1_ref, b1_ref, w2_ref, o_ref):
    dinv = _dinv_of(d0_ref[...], d1_ref[...])
    h = dinv * (p0_ref[0] + p1_ref[0] + z1_ref[...]) + b1_ref[...]
    h = jnp.maximum(h, 0.0)
    y2 = jnp.dot(h, w2_ref[...], preferred_element_type=jnp.float32)
    o_ref[...] = y2 * dinv


def _fin_body(q0_ref, q1_ref, z2_ref, d0_ref, d1_ref, b2_ref, o_ref):
    dinv = _dinv_of(d0_ref[...], d1_ref[...])
    o_ref[...] = dinv * (q0_ref[0] + q1_ref[0] + z2_ref[...]) + b2_ref[...]


def _row_spec(cols):
    return pl.BlockSpec((BR, cols), lambda i: (i, 0))


def _part_spec(cols, which):
    if which == 0:
        return pl.BlockSpec((1, BR, cols), lambda i: (0, i, 0))
    return pl.BlockSpec((1, BR, cols), lambda i: (1, i, 0))


def _full_spec(rows, cols):
    return pl.BlockSpec((rows, cols), lambda i: (0, 0))


def _split_views(flat, nb_fast, nb_slow, k):
    cut = NS * nb_fast * k
    fast = flat[:cut].reshape(NS, nb_fast, k)
    slow = flat[cut:cut + NS * nb_slow * k].reshape(NS, nb_slow, k)
    return fast, slow


def kernel(x, edge_index, W1, b1, W2, b2):
    src = edge_index[0].astype(jnp.int32)
    dst = edge_index[1].astype(jnp.int32)
    e = src.shape[0]

    K1, NBUF1 = 64, 4
    NB1_FAST, NB1_SLOW = 240, 76
    K2, NBUF2 = 128, 4
    NB2_FAST, NB2_SLOW = 132, 28
    nbd = -(-e // (NW * K2))
    slots = NS * (NB2_FAST + NB2_SLOW) * K2
    src_p = jnp.concatenate([src, jnp.zeros((slots - e,), jnp.int32)])
    dst_p = jnp.concatenate([dst, jnp.full((slots - e,), N_NODES, jnp.int32)])
    src_1f, src_1s = _split_views(src_p, NB1_FAST, NB1_SLOW, K1)
    dst_1f, dst_1s = _split_views(dst_p, NB1_FAST, NB1_SLOW, K1)
    src_2f, src_2s = _split_views(src_p, NB2_FAST, NB2_SLOW, K2)
    dst_2f, dst_2s = _split_views(dst_p, NB2_FAST, NB2_SLOW, K2)
    dst_d = dst_p[:NW * nbd * K2].reshape(NW, nbd, K2)

    degp = _make_deg_kernel(nbd, K2)(dst_d)

    grid = (N_NODES // BR,)
    z1 = pl.pallas_call(
        _mm1_body,
        grid=grid,
        in_specs=[_row_spec(IN_DIM), _part_spec(16, 0), _part_spec(16, 1),
                  _full_spec(IN_DIM, HID_DIM)],
        out_specs=_row_spec(HID_DIM),
        out_shape=jax.ShapeDtypeStruct((N_NODES, HID_DIM), jnp.float32),
    )(x, degp, degp, W1)

    agg1 = _make_agg_kernel(NB1_FAST, NB1_SLOW, HID_DIM, K1, NBUF1,
                            nbuf_slow=2, phases=2)(
        z1, src_1f, dst_1f, src_1s, dst_1s)

    z2 = pl.pallas_call(
        _fuse_body,
        grid=grid,
        in_specs=[_part_spec(HID_DIM, 0), _part_spec(HID_DIM, 1),
                  _row_spec(HID_DIM), _part_spec(16, 0), _part_spec(16, 1),
                  _full_spec(1, HID_DIM), _full_spec(HID_DIM, OUT_DIM)],
        out_specs=_row_spec(OUT_DIM),
        out_shape=jax.ShapeDtypeStruct((N_NODES, OUT_DIM), jnp.float32),
    )(agg1, agg1, z1, degp, degp, b1.reshape(1, HID_DIM), W2)

    agg2 = _make_agg_kernel(NB2_FAST, NB2_SLOW, OUT_DIM, K2, NBUF2)(
        z2, src_2f, dst_2f, src_2s, dst_2s)

    out = pl.pallas_call(
        _fin_body,
        grid=grid,
        in_specs=[_part_spec(OUT_DIM, 0), _part_spec(OUT_DIM, 1),
                  _row_spec(OUT_DIM), _part_spec(16, 0), _part_spec(16, 1),
                  _full_spec(1, OUT_DIM)],
        out_specs=_row_spec(OUT_DIM),
        out_shape=jax.ShapeDtypeStruct((N_NODES, OUT_DIM), jnp.float32),
    )(agg2, agg2, z2, degp, degp, b2.reshape(1, OUT_DIM))

    return out


# speedup vs baseline: 1.4511x; 1.1592x over previous
import functools

import jax
import jax.numpy as jnp
from jax import lax
from jax.experimental import pallas as pl
from jax.experimental.pallas import tpu as pltpu
from jax.experimental.pallas import tpu_sc as plsc

N_NODES = 10000
IN_DIM = 128
HID_DIM = 128
OUT_DIM = 16

NC = 2
NS = 16
NW = NC * NS
STRIPE = 640
NPAD = NS * STRIPE
BR = 2000


def _fill(ref, rows, cols, value):
    groups = cols // 16

    def body(t, carry):
        ref[t // groups, pl.ds((t % groups) * 16, 16)] = jnp.full(
            (16,), value, jnp.float32)
        return carry

    lax.fori_loop(0, rows * groups, body, 0)


def _zero_stripe(zref, rows, acc, s):
    d = zref.shape[-1]
    groups = d // 16

    def body(t, carry):
        zref[t // groups, pl.ds((t % groups) * 16, 16)] = jnp.zeros(
            (16,), jnp.float32)
        return carry

    lax.fori_loop(0, rows * groups, body, 0)
    for kk in range(STRIPE // rows):
        pltpu.sync_copy(zref, acc.at[pl.ds(s * STRIPE + kk * rows, rows)])


def _drain_stripe(acc, out_hbm, c, s, rows):
    for kk in range(STRIPE // rows):
        off = s * STRIPE + kk * rows
        pltpu.sync_copy(acc.at[pl.ds(off, rows)],
                        out_hbm.at[c, pl.ds(off, rows)])


def _edge_pipeline(z_hbm, src_v, dst_v, bufs, gsem, acc, n_blocks, nbuf):
    for b in range(nbuf):
        pltpu.async_copy(z_hbm.at[src_v.at[b]], bufs.at[b], gsem[b])

    def body(g, carry):
        for b in range(nbuf):
            j = g * nbuf + b
            pltpu.make_async_copy(
                z_hbm.at[src_v.at[j]], bufs.at[b], gsem[b]).wait()
            pltpu.sync_copy(bufs.at[b], acc.at[dst_v.at[j]], add=True)

            @pl.when(j + nbuf < n_blocks)
            def _():
                pltpu.async_copy(
                    z_hbm.at[src_v.at[j + nbuf]], bufs.at[b], gsem[b])
        return carry

    lax.fori_loop(0, n_blocks // nbuf, body, 0)


def _make_deg_kernel(n_blocks, k):
    mesh = plsc.VectorSubcoreMesh(core_axis_name="c", subcore_axis_name="s")

    @functools.partial(
        pl.kernel,
        mesh=mesh,
        out_type=jax.ShapeDtypeStruct((NC, NPAD, 16), jnp.float32),
        scratch_types=[
            pltpu.VMEM((n_blocks, k), jnp.int32),
            pltpu.VMEM((k, 16), jnp.float32),
            pltpu.VMEM_SHARED((NPAD, 16), jnp.float32),
        ],
        compiler_params=pltpu.CompilerParams(use_tc_tiling_on_sc=False),
    )
    def deg_kernel(dst_hbm, out_hbm, dst_v, ones_v, acc):
        c = lax.axis_index("c")
        s = lax.axis_index("s")
        wid = c * NS + s
        _zero_stripe(ones_v, k, acc, s)
        _fill(ones_v, k, 16, 1.0)
        pltpu.sync_copy(dst_hbm.at[wid], dst_v)
        plsc.subcore_barrier()

        def body(j, carry):
            pltpu.sync_copy(ones_v, acc.at[dst_v.at[j]], add=True)
            return carry

        lax.fori_loop(0, n_blocks, body, 0)
        plsc.subcore_barrier()
        _drain_stripe(acc, out_hbm, c, s, k)

    return deg_kernel


SLOW_CORE = 1


def _make_agg_kernel(nb_fast, nb_slow, d, k, nbuf, nbuf_slow=None, phases=1):
    nbuf_slow = nbuf if nbuf_slow is None else nbuf_slow
    nb_f = nb_fast // phases
    nb_s = nb_slow // phases
    assert nb_fast % phases == 0 and nb_slow % phases == 0
    assert nb_f % nbuf == 0 and nb_s % nbuf_slow == 0 and STRIPE % k == 0
    nb_max = max(nb_f, nb_s)
    mesh = plsc.VectorSubcoreMesh(core_axis_name="c", subcore_axis_name="s")

    @functools.partial(
        pl.kernel,
        mesh=mesh,
        out_type=jax.ShapeDtypeStruct((NC, NPAD, d), jnp.float32),
        scratch_types=[
            pltpu.VMEM((nb_max, k), jnp.int32),
            pltpu.VMEM((nb_max, k), jnp.int32),
            pltpu.VMEM((nbuf, k, d), jnp.float32),
            pltpu.VMEM_SHARED((NPAD, d), jnp.float32),
        ] + [pltpu.SemaphoreType.DMA] * nbuf,
        compiler_params=pltpu.CompilerParams(use_tc_tiling_on_sc=False),
    )
    def agg_kernel(z_hbm, srcf_hbm, dstf_hbm, srcs_hbm, dsts_hbm, out_hbm,
                   src_v, dst_v, bufs, acc, *gsem):
        c = lax.axis_index("c")
        s = lax.axis_index("s")
        _zero_stripe(bufs.at[0], k, acc, s)
        plsc.subcore_barrier()

        for p in range(phases):
            @pl.when(c != SLOW_CORE)
            def _():
                pltpu.sync_copy(srcf_hbm.at[s, pl.ds(p * nb_f, nb_f)],
                                src_v.at[pl.ds(0, nb_f)])
                pltpu.sync_copy(dstf_hbm.at[s, pl.ds(p * nb_f, nb_f)],
                                dst_v.at[pl.ds(0, nb_f)])
                _edge_pipeline(z_hbm, src_v, dst_v, bufs, gsem, acc,
                               nb_f, nbuf)

            @pl.when(c == SLOW_CORE)
            def _():
                pltpu.sync_copy(srcs_hbm.at[s, pl.ds(p * nb_s, nb_s)],
                                src_v.at[pl.ds(0, nb_s)])
                pltpu.sync_copy(dsts_hbm.at[s, pl.ds(p * nb_s, nb_s)],
                                dst_v.at[pl.ds(0, nb_s)])
                _edge_pipeline(z_hbm, src_v, dst_v, bufs, gsem, acc,
                               nb_s, nbuf_slow)

        plsc.subcore_barrier()
        _drain_stripe(acc, out_hbm, c, s, k)

    return agg_kernel


def _dinv_of(d0, d1):
    return lax.rsqrt(1.0 + d0[0][:, :1] + d1[0][:, :1])


def _mm1_body(x_ref, d0_ref, d1_ref, w_ref, o_ref):
    dinv = _dinv_of(d0_ref[...], d1_ref[...])
    y = jnp.dot(x_ref[...], w_ref[...], preferred_element_type=jnp.float32)
    o_ref[...] = y * dinv


def _fuse_body(p0_ref, p1_ref, z1_ref, d0_ref, d1_ref, b1_ref, w2_ref, o_ref):
    dinv = _dinv_of(d0_ref[...], d1_ref[...])
    h = dinv * (p0_ref[0] + p1_ref[0] + z1_ref[...]) + b1_ref[...]
    h = jnp.maximum(h, 0.0)
    y2 = jnp.dot(h, w2_ref[...], preferred_element_type=jnp.float32)
    o_ref[...] = y2 * dinv


def _fin_body(q0_ref, q1_ref, z2_ref, d0_ref, d1_ref, b2_ref, o_ref):
    dinv = _dinv_of(d0_ref[...], d1_ref[...])
    o_ref[...] = dinv * (q0_ref[0] + q1_ref[0] + z2_ref[...]) + b2_ref[...]


def _row_spec(cols):
    return pl.BlockSpec((BR, cols), lambda i: (i, 0))


def _part_spec(cols, which):
    if which == 0:
        return pl.BlockSpec((1, BR, cols), lambda i: (0, i, 0))
    return pl.BlockSpec((1, BR, cols), lambda i: (1, i, 0))


def _full_spec(rows, cols):
    return pl.BlockSpec((rows, cols), lambda i: (0, 0))


def _split_views(flat, nb_fast, nb_slow, k):
    cut = NS * nb_fast * k
    fast = flat[:cut].reshape(NS, nb_fast, k)
    slow = flat[cut:cut + NS * nb_slow * k].reshape(NS, nb_slow, k)
    return fast, slow


def kernel(x, edge_index, W1, b1, W2, b2):
    src = edge_index[0].astype(jnp.int32)
    dst = edge_index[1].astype(jnp.int32)
    e = src.shape[0]

    K1, NBUF1 = 64, 2
    NB1_FAST, NB1_SLOW = 236, 78
    K2, NBUF2 = 128, 4
    NB2_FAST, NB2_SLOW = 132, 28
    nbd = -(-e // (NW * K2))
    slots = NS * (NB2_FAST + NB2_SLOW) * K2
    src_p = jnp.concatenate([src, jnp.zeros((slots - e,), jnp.int32)])
    dst_p = jnp.concatenate([dst, jnp.full((slots - e,), N_NODES, jnp.int32)])
    src_1f, src_1s = _split_views(src_p, NB1_FAST, NB1_SLOW, K1)
    dst_1f, dst_1s = _split_views(dst_p, NB1_FAST, NB1_SLOW, K1)
    src_2f, src_2s = _split_views(src_p, NB2_FAST, NB2_SLOW, K2)
    dst_2f, dst_2s = _split_views(dst_p, NB2_FAST, NB2_SLOW, K2)
    dst_d = dst_p[:NW * nbd * K2].reshape(NW, nbd, K2)

    degp = _make_deg_kernel(nbd, K2)(dst_d)

    grid = (N_NODES // BR,)
    z1 = pl.pallas_call(
        _mm1_body,
        grid=grid,
        in_specs=[_row_spec(IN_DIM), _part_spec(16, 0), _part_spec(16, 1),
                  _full_spec(IN_DIM, HID_DIM)],
        out_specs=_row_spec(HID_DIM),
        out_shape=jax.ShapeDtypeStruct((N_NODES, HID_DIM), jnp.float32),
    )(x, degp, degp, W1)

    agg1 = _make_agg_kernel(NB1_FAST, NB1_SLOW, HID_DIM, K1, NBUF1)(
        z1, src_1f, dst_1f, src_1s, dst_1s)

    z2 = pl.pallas_call(
        _fuse_body,
        grid=grid,
        in_specs=[_part_spec(HID_DIM, 0), _part_spec(HID_DIM, 1),
                  _row_spec(HID_DIM), _part_spec(16, 0), _part_spec(16, 1),
                  _full_spec(1, HID_DIM), _full_spec(HID_DIM, OUT_DIM)],
        out_specs=_row_spec(OUT_DIM),
        out_shape=jax.ShapeDtypeStruct((N_NODES, OUT_DIM), jnp.float32),
    )(agg1, agg1, z1, degp, degp, b1.reshape(1, HID_DIM), W2)

    agg2 = _make_agg_kernel(NB2_FAST, NB2_SLOW, OUT_DIM, K2, NBUF2)(
        z2, src_2f, dst_2f, src_2s, dst_2s)

    out = pl.pallas_call(
        _fin_body,
        grid=grid,
        in_specs=[_part_spec(OUT_DIM, 0), _part_spec(OUT_DIM, 1),
                  _row_spec(OUT_DIM), _part_spec(16, 0), _part_spec(16, 1),
                  _full_spec(1, OUT_DIM)],
        out_specs=_row_spec(OUT_DIM),
        out_shape=jax.ShapeDtypeStruct((N_NODES, OUT_DIM), jnp.float32),
    )(agg2, agg2, z2, degp, degp, b2.reshape(1, OUT_DIM))

    return out
